# baseline (speedup 1.0000x reference)
import functools

import jax
import jax.numpy as jnp
from jax.experimental import pallas as pl
from jax.experimental.pallas import tpu as pltpu

_EMA = 2.0 / (2000 + 1)
_T = 1.0
_C = 1.4426950408889634 / _T
_STORE_ROWS = 1200


def _build_qaug(q):
    b, d = q.shape
    qpad = jnp.concatenate([q, jnp.zeros((b, 128 - d), jnp.float32)], axis=1)
    qt = jnp.transpose(qpad)[:d, :]
    q2col = jnp.sum(q * q, axis=1, keepdims=True)
    q2row = jnp.transpose(jnp.broadcast_to(q2col, (b, 128)))[:1, :]
    ones_row = jnp.ones((1, b), jnp.float32)
    return jnp.concatenate([-2.0 * qt, ones_row, q2row], axis=0)


def _exp_tile(a, qaug):
    a2 = jnp.sum(a * a, axis=1, keepdims=True)
    aug = jnp.concatenate([a, a2, jnp.ones_like(a2)], axis=1)
    d2 = jnp.dot(aug, qaug,
                 preferred_element_type=jnp.float32)
    d2 = jnp.maximum(d2, 1e-12)
    return jnp.exp2((-_C * d2) * jax.lax.rsqrt(d2))


def _stats_body(q_ref, a_ref, z_ref, s_ref, qaug_sc, *, ms):
    i = pl.program_id(0)

    @pl.when(i == 0)
    def _prep():
        qaug_sc[...] = _build_qaug(q_ref[...])

    s = _exp_tile(a_ref[...], qaug_sc[...])
    s16 = s.astype(jnp.bfloat16)
    s_ref[...] = s16[:ms]
    ones_row = jnp.ones((1, s.shape[0]), jnp.bfloat16)
    part = jnp.dot(ones_row, s16,
                   preferred_element_type=jnp.float32)

    @pl.when(i == 0)
    def _init():
        z_ref[...] = part

    @pl.when(i > 0)
    def _acc():
        z_ref[...] += part


def _update_body(q_ref, qc_ref, z_ref, s_ref, a_ref, mc_ref, out_ref,
                 qaug_sc, qall_sc, *, d, nc, ms):
    i = pl.program_id(0)

    @pl.when(i == 0)
    def _prep():
        q = q_ref[...]
        b = q.shape[0]
        qaug_sc[...] = _build_qaug(q)
        inv_row = _EMA / z_ref[...]
        inv_col = jnp.transpose(
            jnp.broadcast_to(inv_row, (128, b)))[:, :1]
        qall = jnp.concatenate(
            [q, qc_ref[...], jnp.ones((b, 1), jnp.float32)], axis=1)
        qall_sc[...] = (qall * inv_col).astype(jnp.bfloat16)

    a = a_ref[...]
    qall = qall_sc[...]
    p_top = jnp.dot(s_ref[...], qall,
                    preferred_element_type=jnp.float32)
    s_bot = _exp_tile(a[ms:], qaug_sc[...]).astype(jnp.bfloat16)
    p_bot = jnp.dot(s_bot, qall,
                    preferred_element_type=jnp.float32)
    am = jnp.concatenate([a, mc_ref[...]], axis=1)
    out_ref[:ms] = am[:ms] * (1.0 - p_top[:, d + nc:]) + p_top[:, :d + nc]
    out_ref[ms:] = am[ms:] * (1.0 - p_bot[:, d + nc:]) + p_bot[:, :d + nc]


@jax.jit
def kernel(query_address, query_content, A, Mc):
    b, d = query_address.shape
    m = A.shape[0]
    nc = query_content.shape[1]

    mt = 2000 if m % 2000 == 0 else (1000 if m % 1000 == 0 else m)
    nt = m // mt
    ms = _STORE_ROWS if mt == 2000 else max(8, (mt * 23 // 100) // 8 * 8)

    full = lambda shape: pl.BlockSpec(shape, lambda i: (0, 0))
    z, s16 = pl.pallas_call(
        functools.partial(_stats_body, ms=ms),
        grid=(nt,),
        in_specs=[full((b, d)),
                  pl.BlockSpec((mt, d), lambda i: (i, 0))],
        out_specs=[full((1, b)), pl.BlockSpec((ms, b), lambda i: (i, 0))],
        out_shape=[jax.ShapeDtypeStruct((1, b), jnp.float32),
                   jax.ShapeDtypeStruct((nt * ms, b), jnp.bfloat16)],
        scratch_shapes=[pltpu.VMEM((d + 2, b), jnp.float32)],
    )(query_address, A)

    out = pl.pallas_call(
        functools.partial(_update_body, d=d, nc=nc, ms=ms),
        grid=(nt,),
        in_specs=[full((b, d)), full((b, nc)), full((1, b)),
                  pl.BlockSpec((ms, b), lambda i: (i, 0)),
                  pl.BlockSpec((mt, d), lambda i: (i, 0)),
                  pl.BlockSpec((mt, nc), lambda i: (i, 0))],
        out_specs=pl.BlockSpec((mt, d + nc), lambda i: (i, 0)),
        out_shape=jax.ShapeDtypeStruct((m, d + nc), jnp.float32),
        scratch_shapes=[pltpu.VMEM((d + 2, b), jnp.float32),
                        pltpu.VMEM((b, d + nc + 1), jnp.bfloat16)],
    )(query_address, query_content, z, s16, A, Mc)
    return out


# speedup vs baseline: 1.0495x; 1.0495x over previous
import functools

import jax
import jax.numpy as jnp
from jax.experimental import pallas as pl
from jax.experimental.pallas import tpu as pltpu

_EMA = 2.0 / (2000 + 1)
_T = 1.0
_C = 1.4426950408889634 / _T
_STORE_ROWS = 880


def _build_qaug(q):
    b, d = q.shape
    qpad = jnp.concatenate([q, jnp.zeros((b, 128 - d), jnp.float32)], axis=1)
    qt = jnp.transpose(qpad)[:d, :]
    q2col = jnp.sum(q * q, axis=1, keepdims=True)
    q2row = jnp.transpose(jnp.broadcast_to(q2col, (b, 128)))[:1, :]
    ones_row = jnp.ones((1, b), jnp.float32)
    return jnp.concatenate([-2.0 * qt, ones_row, q2row], axis=0)


def _exp_tile(a, qaug):
    a2 = jnp.sum(a * a, axis=1, keepdims=True)
    aug = jnp.concatenate([a, a2, jnp.ones_like(a2)], axis=1)
    d2 = jnp.dot(aug, qaug,
                 preferred_element_type=jnp.float32)
    d2 = jnp.maximum(d2, 1e-12)
    return jnp.exp2((-_C * d2) * jax.lax.rsqrt(d2))


def _stats_body(q_ref, a_ref, z_ref, s_ref, qaug_sc, *, ms):
    i = pl.program_id(0)

    @pl.when(i == 0)
    def _prep():
        qaug_sc[...] = _build_qaug(q_ref[...])

    s = _exp_tile(a_ref[...], qaug_sc[...])
    s16 = s.astype(jnp.bfloat16)
    s_ref[...] = s16[:ms]
    ones_row = jnp.ones((1, s.shape[0]), jnp.bfloat16)
    part = jnp.dot(ones_row, s16,
                   preferred_element_type=jnp.float32)

    @pl.when(i == 0)
    def _init():
        z_ref[...] = part

    @pl.when(i > 0)
    def _acc():
        z_ref[...] += part


def _update_body(q_ref, qc_ref, z_ref, s_ref, a_ref, mc_ref, out_ref,
                 qaug_sc, qall_sc, *, d, nc, ms):
    i = pl.program_id(0)

    @pl.when(i == 0)
    def _prep():
        q = q_ref[...]
        b = q.shape[0]
        qaug_sc[...] = _build_qaug(q)
        inv_row = _EMA / z_ref[...]
        inv_col = jnp.transpose(
            jnp.broadcast_to(inv_row, (128, b)))[:, :1]
        qall = jnp.concatenate(
            [q, qc_ref[...], jnp.ones((b, 1), jnp.float32)], axis=1)
        qall_sc[...] = (qall * inv_col).astype(jnp.bfloat16)

    a = a_ref[...]
    qall = qall_sc[...]
    p_top = jnp.dot(s_ref[...], qall,
                    preferred_element_type=jnp.float32)
    s_bot = _exp_tile(a[ms:], qaug_sc[...]).astype(jnp.bfloat16)
    p_bot = jnp.dot(s_bot, qall,
                    preferred_element_type=jnp.float32)
    am = jnp.concatenate([a, mc_ref[...]], axis=1)
    out_ref[:ms] = am[:ms] * (1.0 - p_top[:, d + nc:]) + p_top[:, :d + nc]
    out_ref[ms:] = am[ms:] * (1.0 - p_bot[:, d + nc:]) + p_bot[:, :d + nc]


@jax.jit
def kernel(query_address, query_content, A, Mc):
    b, d = query_address.shape
    m = A.shape[0]
    nc = query_content.shape[1]

    mt = 2000 if m % 2000 == 0 else (1000 if m % 1000 == 0 else m)
    nt = m // mt
    ms = _STORE_ROWS if mt == 2000 else max(8, (mt * 23 // 100) // 8 * 8)

    full = lambda shape: pl.BlockSpec(shape, lambda i: (0, 0))
    z, s16 = pl.pallas_call(
        functools.partial(_stats_body, ms=ms),
        grid=(nt,),
        in_specs=[full((b, d)),
                  pl.BlockSpec((mt, d), lambda i: (i, 0))],
        out_specs=[full((1, b)), pl.BlockSpec((ms, b), lambda i: (i, 0))],
        out_shape=[jax.ShapeDtypeStruct((1, b), jnp.float32),
                   jax.ShapeDtypeStruct((nt * ms, b), jnp.bfloat16)],
        scratch_shapes=[pltpu.VMEM((d + 2, b), jnp.float32)],
    )(query_address, A)

    out = pl.pallas_call(
        functools.partial(_update_body, d=d, nc=nc, ms=ms),
        grid=(nt,),
        in_specs=[full((b, d)), full((b, nc)), full((1, b)),
                  pl.BlockSpec((ms, b), lambda i: (i, 0)),
                  pl.BlockSpec((mt, d), lambda i: (i, 0)),
                  pl.BlockSpec((mt, nc), lambda i: (i, 0))],
        out_specs=pl.BlockSpec((mt, d + nc), lambda i: (i, 0)),
        out_shape=jax.ShapeDtypeStruct((m, d + nc), jnp.float32),
        scratch_shapes=[pltpu.VMEM((d + 2, b), jnp.float32),
                        pltpu.VMEM((b, d + nc + 1), jnp.bfloat16)],
    )(query_address, query_content, z, s16, A, Mc)
    return out
